# baseline (speedup 1.0000x reference)
import jax
import jax.numpy as jnp
from jax import lax
from jax.experimental import pallas as pl
from jax.experimental.pallas import tpu as pltpu
from jax.experimental.pallas import tpu_sc as plsc

_E = 8
_L = 16


def _pool_body(x_ref, wg_ref, logits_ref):
    xb = x_ref[0]
    xg = jnp.mean(xb, axis=0, keepdims=True)
    lg = jnp.dot(xg, wg_ref[...],
                 preferred_element_type=jnp.float32)
    logits_ref[...] = jnp.clip(lg, -50.0, 50.0)[None]


def _sc_route(logits_hbm, eidx_hbm, gvals_hbm, gates_hbm,
              lg_v, ei_v, gv_v, ga_v):
    info = plsc.get_sparse_core_info()
    wid = lax.axis_index("s") * info.num_cores + lax.axis_index("c")

    @pl.when(wid == 0)
    def _():
        pltpu.sync_copy(logits_hbm, lg_v)
        B = lg_v.shape[1]
        for c in range(B // _L):
            sl = pl.ds(c * _L, _L)
            le = [lg_v[e, sl] for e in range(_E)]
            v0 = le[0]
            for e in range(1, _E):
                v0 = jnp.maximum(v0, le[e])
            e0 = jnp.full((_L,), _E, jnp.int32)
            for e in reversed(range(_E)):
                e0 = jnp.where(le[e] == v0, e, e0)
            neg = jnp.full((_L,), -jnp.inf, jnp.float32)
            m = [jnp.where(e0 == e, neg, le[e]) for e in range(_E)]
            v1 = m[0]
            for e in range(1, _E):
                v1 = jnp.maximum(v1, m[e])
            e1 = jnp.full((_L,), _E, jnp.int32)
            for e in reversed(range(_E)):
                e1 = jnp.where(m[e] == v1, e, e1)
            t = jnp.exp(v1 - v0)
            g0 = 1.0 / (1.0 + t)
            g1 = t / (1.0 + t)
            ei_v[0, sl] = e0
            ei_v[1, sl] = e1
            gv_v[0, sl] = g0
            gv_v[1, sl] = g1
            for e in range(_E):
                ga_v[e, sl] = (jnp.where(e0 == e, g0, 0.0)
                               + jnp.where(e1 == e, g1, 0.0))

        pltpu.sync_copy(ei_v, eidx_hbm)
        pltpu.sync_copy(gv_v, gvals_hbm)
        pltpu.sync_copy(ga_v, gates_hbm)


def _loss_body(gates_ref, loss_ref):
    g = gates_ref[...]
    n = float(_E)
    eps = 1e-10

    def cv_sq(v):
        m = jnp.sum(v) / n
        var = jnp.sum((v - m) ** 2) / (n - 1.0)
        return var / (m * m + eps)

    imp = jnp.sum(g, axis=1, keepdims=True)
    load = jnp.sum((g > 0.0).astype(jnp.float32), axis=1, keepdims=True)
    loss = cv_sq(imp) + cv_sq(load)
    loss_ref[...] = jnp.clip(loss, 0.0, 1000.0) * jnp.ones((1, 1), jnp.float32)


def _expert_body(eidx_ref, gvals_ref, x_ref, W1_ref, b1_ref, W2_ref, b2_ref,
                 y_ref):
    b = pl.program_id(0)
    xb = x_ref[0]
    e0 = eidx_ref[0, b]
    e1 = eidx_ref[1, b]
    g0 = gvals_ref[0, b]
    g1 = gvals_ref[1, b]

    def expert(e):
        h = jnp.dot(xb, W1_ref[e], preferred_element_type=jnp.float32)
        h = h + b1_ref[e][None, :]
        h = 0.5 * h * (1.0 + jax.lax.erf(h * 0.7071067811865476))
        o = jnp.dot(h, W2_ref[e], preferred_element_type=jnp.float32)
        return o + b2_ref[e][None, :]

    y_ref[0] = g0 * expert(e0) + g1 * expert(e1)


def kernel(x, w_gate, W1, b1, W2, b2):
    B, H, W, C = x.shape
    T = H * W
    E = w_gate.shape[1]
    x_flat = x.reshape(B, T, C)

    logits = pl.pallas_call(
        _pool_body,
        grid=(B,),
        in_specs=[
            pl.BlockSpec((1, T, C), lambda b: (b, 0, 0)),
            pl.BlockSpec((C, E), lambda b: (0, 0)),
        ],
        out_specs=pl.BlockSpec((1, 1, E), lambda b: (b, 0, 0)),
        out_shape=jax.ShapeDtypeStruct((B, 1, E), jnp.float32),
        compiler_params=pltpu.CompilerParams(
            dimension_semantics=("parallel",),
        ),
    )(x_flat, w_gate)

    logits_t = logits.reshape(B, E).T

    mesh = plsc.VectorSubcoreMesh(core_axis_name="c", subcore_axis_name="s")
    route = pl.kernel(
        _sc_route,
        mesh=mesh,
        out_type=[
            jax.ShapeDtypeStruct((2, B), jnp.int32),
            jax.ShapeDtypeStruct((2, B), jnp.float32),
            jax.ShapeDtypeStruct((E, B), jnp.float32),
        ],
        scratch_types=[
            pltpu.VMEM((E, B), jnp.float32),
            pltpu.VMEM((2, B), jnp.int32),
            pltpu.VMEM((2, B), jnp.float32),
            pltpu.VMEM((E, B), jnp.float32),
        ],
    )
    eidx_t, gvals_t, gates_t = route(logits_t)

    y_flat = pl.pallas_call(
        _expert_body,
        grid=(B,),
        in_specs=[
            pl.BlockSpec(memory_space=pltpu.SMEM),
            pl.BlockSpec(memory_space=pltpu.SMEM),
            pl.BlockSpec((1, T, C), lambda b: (b, 0, 0)),
            pl.BlockSpec(W1.shape, lambda b: (0, 0, 0)),
            pl.BlockSpec(b1.shape, lambda b: (0, 0)),
            pl.BlockSpec(W2.shape, lambda b: (0, 0, 0)),
            pl.BlockSpec(b2.shape, lambda b: (0, 0)),
        ],
        out_specs=pl.BlockSpec((1, T, C), lambda b: (b, 0, 0)),
        out_shape=jax.ShapeDtypeStruct((B, T, C), jnp.float32),
        compiler_params=pltpu.CompilerParams(
            dimension_semantics=("parallel",),
        ),
    )(eidx_t, gvals_t, x_flat, W1, b1, W2, b2)

    loss = pl.pallas_call(
        _loss_body,
        out_shape=jax.ShapeDtypeStruct((1, 1), jnp.float32),
    )(gates_t)

    return y_flat.reshape(B, H, W, C), loss[0, 0]


# speedup vs baseline: 1.6736x; 1.6736x over previous
import jax
import jax.numpy as jnp
from jax import lax
from jax.experimental import pallas as pl
from jax.experimental.pallas import tpu as pltpu
from jax.experimental.pallas import tpu_sc as plsc

_E = 8
_L = 16


_G = 8


def _pool_body(x_ref, wg_ref, logits_ref):
    xg = jnp.mean(x_ref[...], axis=1)
    lg = jnp.dot(xg, wg_ref[...],
                 preferred_element_type=jnp.float32)
    logits_ref[...] = jnp.clip(lg, -50.0, 50.0)[:, None, :]


def _sc_route(logits_hbm, eidx_hbm, gvals_hbm, gates_hbm,
              lg_v, ei_v, gv_v, ga_v):
    info = plsc.get_sparse_core_info()
    wid = lax.axis_index("s") * info.num_cores + lax.axis_index("c")

    @pl.when(wid == 0)
    def _():
        pltpu.sync_copy(logits_hbm, lg_v)
        B = lg_v.shape[1]
        for c in range(B // _L):
            sl = pl.ds(c * _L, _L)
            le = [lg_v[e, sl] for e in range(_E)]
            v0 = le[0]
            for e in range(1, _E):
                v0 = jnp.maximum(v0, le[e])
            e0 = jnp.full((_L,), _E, jnp.int32)
            for e in reversed(range(_E)):
                e0 = jnp.where(le[e] == v0, e, e0)
            neg = jnp.full((_L,), -jnp.inf, jnp.float32)
            m = [jnp.where(e0 == e, neg, le[e]) for e in range(_E)]
            v1 = m[0]
            for e in range(1, _E):
                v1 = jnp.maximum(v1, m[e])
            e1 = jnp.full((_L,), _E, jnp.int32)
            for e in reversed(range(_E)):
                e1 = jnp.where(m[e] == v1, e, e1)
            t = jnp.exp(v1 - v0)
            g0 = 1.0 / (1.0 + t)
            g1 = t / (1.0 + t)
            ei_v[0, sl] = e0
            ei_v[1, sl] = e1
            gv_v[0, sl] = g0
            gv_v[1, sl] = g1
            for e in range(_E):
                ga_v[e, sl] = (jnp.where(e0 == e, g0, 0.0)
                               + jnp.where(e1 == e, g1, 0.0))

        pltpu.sync_copy(ei_v, eidx_hbm)
        pltpu.sync_copy(gv_v, gvals_hbm)
        pltpu.sync_copy(ga_v, gates_hbm)


def _loss_body(gates_ref, loss_ref):
    g = gates_ref[...]
    n = float(_E)
    eps = 1e-10

    def cv_sq(v):
        m = jnp.sum(v) / n
        var = jnp.sum((v - m) ** 2) / (n - 1.0)
        return var / (m * m + eps)

    imp = jnp.sum(g, axis=1, keepdims=True)
    load = jnp.sum((g > 0.0).astype(jnp.float32), axis=1, keepdims=True)
    loss = cv_sq(imp) + cv_sq(load)
    loss_ref[...] = jnp.clip(loss, 0.0, 1000.0) * jnp.ones((1, 1), jnp.float32)


def _expert_body(eidx_ref, gvals_ref, x_ref, W1_ref, b1_ref, W2_ref, b2_ref,
                 y_ref):
    base = pl.program_id(0) * _G

    def expert(xb, e):
        h = jnp.dot(xb, W1_ref[e], preferred_element_type=jnp.float32)
        h = h + b1_ref[e][None, :]
        h = 0.5 * h * (1.0 + jax.lax.erf(h * 0.7071067811865476))
        o = jnp.dot(h, W2_ref[e], preferred_element_type=jnp.float32)
        return o + b2_ref[e][None, :]

    for i in range(_G):
        b = base + i
        xb = x_ref[i]
        e0 = eidx_ref[0, b]
        e1 = eidx_ref[1, b]
        g0 = gvals_ref[0, b]
        g1 = gvals_ref[1, b]
        y_ref[i] = g0 * expert(xb, e0) + g1 * expert(xb, e1)


def kernel(x, w_gate, W1, b1, W2, b2):
    B, H, W, C = x.shape
    T = H * W
    E = w_gate.shape[1]
    x_flat = x.reshape(B, T, C)

    logits = pl.pallas_call(
        _pool_body,
        grid=(B // _G,),
        in_specs=[
            pl.BlockSpec((_G, T, C), lambda b: (b, 0, 0)),
            pl.BlockSpec((C, E), lambda b: (0, 0)),
        ],
        out_specs=pl.BlockSpec((_G, 1, E), lambda b: (b, 0, 0)),
        out_shape=jax.ShapeDtypeStruct((B, 1, E), jnp.float32),
        compiler_params=pltpu.CompilerParams(
            dimension_semantics=("parallel",),
        ),
    )(x_flat, w_gate)

    logits_t = logits.reshape(B, E).T

    mesh = plsc.VectorSubcoreMesh(core_axis_name="c", subcore_axis_name="s")
    route = pl.kernel(
        _sc_route,
        mesh=mesh,
        out_type=[
            jax.ShapeDtypeStruct((2, B), jnp.int32),
            jax.ShapeDtypeStruct((2, B), jnp.float32),
            jax.ShapeDtypeStruct((E, B), jnp.float32),
        ],
        scratch_types=[
            pltpu.VMEM((E, B), jnp.float32),
            pltpu.VMEM((2, B), jnp.int32),
            pltpu.VMEM((2, B), jnp.float32),
            pltpu.VMEM((E, B), jnp.float32),
        ],
    )
    eidx_t, gvals_t, gates_t = route(logits_t)

    y_flat = pl.pallas_call(
        _expert_body,
        grid=(B // _G,),
        in_specs=[
            pl.BlockSpec(memory_space=pltpu.SMEM),
            pl.BlockSpec(memory_space=pltpu.SMEM),
            pl.BlockSpec((_G, T, C), lambda b: (b, 0, 0)),
            pl.BlockSpec(W1.shape, lambda b: (0, 0, 0)),
            pl.BlockSpec(b1.shape, lambda b: (0, 0)),
            pl.BlockSpec(W2.shape, lambda b: (0, 0, 0)),
            pl.BlockSpec(b2.shape, lambda b: (0, 0)),
        ],
        out_specs=pl.BlockSpec((_G, T, C), lambda b: (b, 0, 0)),
        out_shape=jax.ShapeDtypeStruct((B, T, C), jnp.float32),
        compiler_params=pltpu.CompilerParams(
            dimension_semantics=("parallel",),
        ),
    )(eidx_t, gvals_t, x_flat, W1, b1, W2, b2)

    loss = pl.pallas_call(
        _loss_body,
        out_shape=jax.ShapeDtypeStruct((1, 1), jnp.float32),
    )(gates_t)

    return y_flat.reshape(B, H, W, C), loss[0, 0]
